# Initial kernel scaffold; baseline (speedup 1.0000x reference)
import jax
import jax.numpy as jnp
from jax.experimental import pallas as pl


def kernel(features1, features2):
    raise NotImplementedError("write your pallas kernel here")


# speedup vs baseline: 15.0839x; 15.0839x over previous
import functools

import jax
import jax.numpy as jnp
from jax.experimental import pallas as pl

_C = 192
_H = 32
_W = 32
_N = _H * _W
_NPAD = 1152
_OH = 30
_NEG = -3.0e38


def _match_kernel(f1_ref, f2_ref, fw_ref, fh_ref, sv_ref):
    f1 = f1_ref[0]
    f2 = f2_ref[0]

    n1 = jnp.sqrt(jnp.sum(f1 * f1, axis=0, keepdims=True))
    fi = f1 / jnp.maximum(n1, 1e-12)
    n2 = jnp.sqrt(jnp.sum(f2 * f2, axis=0, keepdims=True))
    fr = f2 / jnp.maximum(n2, 1e-12)

    sqi = jnp.sum(fi * fi, axis=0, keepdims=True)
    sqr = jnp.sum(fr * fr, axis=0, keepdims=True)

    rn2 = jnp.zeros((1, _N), jnp.float32)
    in2 = jnp.zeros((1, _N), jnp.float32)
    for di in range(3):
        for dj in range(3):
            o = di * _W + dj
            rn2 = rn2 + jax.lax.slice(sqr, (0, o), (1, o + _N))
            in2 = in2 + jax.lax.slice(sqi, (0, o), (1, o + _N))
    rn = jnp.sqrt(rn2) + 1e-5

    acc = jnp.zeros((_N, _N), jnp.float32)
    for di in range(3):
        for dj in range(3):
            o = di * _W + dj
            a = jax.lax.slice(fi, (0, o), (_C, o + _N))
            b = jax.lax.slice(fr, (0, o), (_C, o + _N)) / rn
            acc = acc + jax.lax.dot_general(
                a, b, (((0,), (0,)), ((), ())),
                preferred_element_type=jnp.float32)

    col = jax.lax.broadcasted_iota(jnp.int32, (1, _N), 1)
    colvalid = ((col % _W) < _OH) & ((col // _W) < _OH)
    corr = jnp.where(colvalid, acc, _NEG)

    maxval = jnp.max(corr, axis=1, keepdims=True)
    lane = jax.lax.broadcasted_iota(jnp.int32, (_N, _N), 1)
    idx = jnp.min(jnp.where(corr == maxval, lane, jnp.int32(1 << 30)),
                  axis=1, keepdims=True)

    r0 = jax.lax.broadcasted_iota(jnp.int32, (_N, _N), 0)
    c0 = jax.lax.broadcasted_iota(jnp.int32, (_N, _N), 1)
    ident = (r0 == c0).astype(jnp.float32)
    in2col = jax.lax.dot_general(
        ident, in2, (((1,), (1,)), ((), ())),
        preferred_element_type=jnp.float32)

    sim = maxval / (jnp.sqrt(in2col) + 1e-5)

    row = jax.lax.broadcasted_iota(jnp.int32, (_N, 1), 0)
    qx = row % _W
    qy = row // _W
    rvalid = (qx < _OH) & (qy < _OH)
    fx = (idx % _W - qx).astype(jnp.float32)
    fy = (idx // _W - qy).astype(jnp.float32)

    fw_ref[0] = jnp.where(rvalid, fx, 0.0)
    fh_ref[0] = jnp.where(rvalid, fy, 0.0)
    sv_ref[0] = jnp.where(rvalid, sim, 0.0)


@jax.jit
def kernel(features1, features2):
    b = features1.shape[0]
    f1 = jnp.pad(features1.reshape(b, _C, _N), ((0, 0), (0, 0), (0, _NPAD - _N)))
    f2 = jnp.pad(features2.reshape(b, _C, _N), ((0, 0), (0, 0), (0, _NPAD - _N)))

    out_shape = jax.ShapeDtypeStruct((b, _N, 1), jnp.float32)
    fw, fh, sv = pl.pallas_call(
        _match_kernel,
        grid=(b,),
        in_specs=[
            pl.BlockSpec((1, _C, _NPAD), lambda i: (i, 0, 0)),
            pl.BlockSpec((1, _C, _NPAD), lambda i: (i, 0, 0)),
        ],
        out_specs=[
            pl.BlockSpec((1, _N, 1), lambda i: (i, 0, 0)),
            pl.BlockSpec((1, _N, 1), lambda i: (i, 0, 0)),
            pl.BlockSpec((1, _N, 1), lambda i: (i, 0, 0)),
        ],
        out_shape=[out_shape, out_shape, out_shape],
    )(f1, f2)

    fw = fw.reshape(b, _H, _W)
    fh = fh.reshape(b, _H, _W)
    sv = sv.reshape(b, _H, _W)

    pre_flow = jnp.stack([fw, fh], axis=-1)
    pre_similarity = jnp.pad(sv[:, :_OH, :_OH],
                             ((0, 0), (1, 1), (1, 1)))[:, None]
    shifted = [
        jnp.pad(pre_flow[:, :_H - i, :_W - j, :],
                ((0, 0), (i, 0), (j, 0), (0, 0)))
        for i in range(3) for j in range(3)
    ]
    pre_offset = jnp.stack(shifted, axis=1)
    return (pre_flow, pre_offset, pre_similarity)
